# baseline (speedup 1.0000x reference)
import functools
import jax, jax.numpy as jnp
from jax import lax
from jax.experimental import pallas as pl
from jax.experimental.pallas import tpu as pltpu
from jax.experimental.pallas import tpu_sc as plsc

G = 100
NPG = 100
NP = 128
N = G * NPG
D = 128
NHID = 128
K1, K1P = 50, 64
K2, K2P = 25, 32


def _eye(n):
    r = jax.lax.broadcasted_iota(jnp.int32, (n, n), 0)
    c = jax.lax.broadcasted_iota(jnp.int32, (n, n), 1)
    return jnp.where(r == c, 1.0, 0.0).astype(jnp.float32)


def _gcn_conv(A, X, W, brow, n, nreal):
    Ah = A + _eye(n)
    dc = jnp.sum(Ah, axis=1, keepdims=True)
    dr = jnp.sum(Ah, axis=0, keepdims=True)
    disc = 1.0 / jnp.sqrt(jnp.clip(dc, 1e-6))
    disr = 1.0 / jnp.sqrt(jnp.clip(dr, 1e-6))
    An = (jnp.broadcast_to(disc, (n, n)) * Ah) * jnp.broadcast_to(disr, (n, n))
    M = jax.lax.dot(An, X, preferred_element_type=jnp.float32)
    Y = jnp.maximum(jax.lax.dot(M, W, preferred_element_type=jnp.float32)
                    + jnp.broadcast_to(brow, (n, NHID)), 0.0)
    ri = jax.lax.broadcasted_iota(jnp.int32, (n, NHID), 0)
    return jnp.where(ri < nreal, Y, 0.0)


def _score_row(A, X, n):
    degc = jnp.clip(jnp.sum(A, axis=1, keepdims=True), 1.0)
    agg = jax.lax.dot(A, X, preferred_element_type=jnp.float32) \
        / jnp.broadcast_to(degc, (n, NHID))
    sc = jnp.sum(jnp.abs(X - agg), axis=1, keepdims=True)
    scT = jnp.transpose(jnp.broadcast_to(sc, (n, NHID)))
    return scT[0:1, :]


def _topk_ranks(scores, n, nreal, k):
    col = jax.lax.broadcasted_iota(jnp.int32, (G, n), 1)
    sc = jnp.where(col < nreal, scores, -1.0)
    rank0 = jnp.full((G, n), 999.0, dtype=jnp.float32)

    def body(j, carry):
        sc, rank = carry
        mx = jnp.max(sc, axis=1, keepdims=True)
        cand = sc == mx
        am = jnp.min(jnp.where(cand, col, n), axis=1, keepdims=True)
        oh = col == am
        rank = jnp.where(oh, j.astype(jnp.float32), rank)
        return jnp.where(oh, -2.0, sc), rank

    _, rank = jax.lax.fori_loop(0, k, body, (sc, rank0))
    return rank


def _P_from_rank(rankrow, kp, n):
    rk = jnp.broadcast_to(rankrow, (kp, n)).astype(jnp.int32)
    rowi = jax.lax.broadcasted_iota(jnp.int32, (kp, n), 0)
    return jnp.where(rk == rowi, 1.0, 0.0).astype(jnp.float32)


def _mega_body(A0_ref, X0_ref, W1_ref, b1_ref, W2_ref, b2_ref, W3_ref, b3_ref,
               X1_ref, X2_ref, X3_ref, sum_ref,
               sc1_ref, rk1_ref, Xp1_ref, A1_ref, sc2_ref, rk2_ref,
               Xp2_ref, A2_ref):
    W1, b1 = W1_ref[...], b1_ref[...]
    W2, b2 = W2_ref[...], b2_ref[...]
    W3, b3 = W3_ref[...], b3_ref[...]
    HI = jax.lax.Precision.HIGHEST

    def stage_a(g, c):
        A = A0_ref[g]
        X1 = _gcn_conv(A, X0_ref[g], W1, b1, NP, NPG)
        X1_ref[g] = X1
        sc1_ref[g] = _score_row(A, X1, NP)
        return c

    jax.lax.fori_loop(0, G, stage_a, 0)
    rk1_ref[...] = _topk_ranks(sc1_ref[...].reshape(G, NP), NP, NPG, K1) \
        .reshape(G, 1, NP)

    def stage_b(g, c):
        P = _P_from_rank(rk1_ref[g], K1P, NP)
        A = A0_ref[g]
        Xp = jax.lax.dot(P, X1_ref[g], precision=HI,
                         preferred_element_type=jnp.float32)
        Xp1_ref[g] = Xp
        Ar = jax.lax.dot(P, A, preferred_element_type=jnp.float32)
        A1 = jax.lax.dot_general(Ar, P, (((1,), (1,)), ((), ())),
                                 preferred_element_type=jnp.float32)
        A1_ref[g] = A1
        X2 = _gcn_conv(A1, Xp, W2, b2, K1P, K1)
        X2_ref[g] = X2
        sc2_ref[g] = _score_row(A1, X2, K1P)
        return c

    jax.lax.fori_loop(0, G, stage_b, 0)
    rk2_ref[...] = _topk_ranks(sc2_ref[...].reshape(G, K1P), K1P, K1, K2) \
        .reshape(G, 1, K1P)

    def stage_c(g, c):
        P = _P_from_rank(rk2_ref[g], K2P, K1P)
        Xp = jax.lax.dot(P, X2_ref[g], precision=HI,
                         preferred_element_type=jnp.float32)
        Xp2_ref[g] = Xp
        Ar = jax.lax.dot(P, A1_ref[g], preferred_element_type=jnp.float32)
        A2 = jax.lax.dot_general(Ar, P, (((1,), (1,)), ((), ())),
                                 preferred_element_type=jnp.float32)
        A2_ref[g] = A2
        X3 = _gcn_conv(A2, Xp, W3, b3, K2P, K2)
        X3_ref[g] = X3

        Xp1 = Xp1_ref[g]
        mx1 = jnp.max(Xp1, axis=0, keepdims=True)
        mn1 = jnp.sum(Xp1, axis=0, keepdims=True) / float(K1)
        mx2 = jnp.max(Xp, axis=0, keepdims=True)
        mn2 = jnp.sum(Xp, axis=0, keepdims=True) / float(K2)
        mx3 = jnp.max(X3, axis=0, keepdims=True)
        mn3 = jnp.sum(X3, axis=0, keepdims=True) / float(K2)
        r = jnp.maximum
        smax = r(mx1, 0.) + r(mx2, 0.) + r(mx3, 0.)
        smean = r(mn1, 0.) + r(mn2, 0.) + r(mn3, 0.)
        sum_ref[g] = jnp.concatenate([smax, smean], axis=1)
        return c

    jax.lax.fori_loop(0, G, stage_c, 0)


def _megakernel(A0p, X0p, W1, b1, W2, b2, W3, b3):
    f32 = jnp.float32
    return pl.pallas_call(
        _mega_body,
        out_shape=(jax.ShapeDtypeStruct((G, NP, NHID), f32),
                   jax.ShapeDtypeStruct((G, K1P, NHID), f32),
                   jax.ShapeDtypeStruct((G, K2P, NHID), f32),
                   jax.ShapeDtypeStruct((G, 1, 2 * NHID), f32)),
        scratch_shapes=[pltpu.VMEM((G, 1, NP), f32),
                        pltpu.VMEM((G, 1, NP), f32),
                        pltpu.VMEM((G, K1P, NHID), f32),
                        pltpu.VMEM((G, K1P, K1P), f32),
                        pltpu.VMEM((G, 1, K1P), f32),
                        pltpu.VMEM((G, 1, K1P), f32),
                        pltpu.VMEM((G, K2P, NHID), f32),
                        pltpu.VMEM((G, K2P, K2P), f32)],
    )(A0p, X0p, W1, b1.reshape(1, NHID), W2, b2.reshape(1, NHID),
      W3, b3.reshape(1, NHID))


E = 320000
GPT = 4
CH = 8000
ROWS = GPT * NP


def _adj_sc_body(src_hbm, dst_hbm, out_hbm, acc, sbuf, dbuf):
    wid = lax.axis_index("s") * 2 + lax.axis_index("c")
    zero16 = jnp.zeros((16,), jnp.float32)
    ones16 = jnp.ones((16,), jnp.float32)

    def zbody(i, c):
        acc[pl.ds(i * 16, 16)] = zero16
        return c

    lax.fori_loop(0, ROWS * 8, zbody, 0)

    gbase = wid * GPT

    def chunk(c, carry):
        pltpu.sync_copy(src_hbm.at[pl.ds(c * CH, CH)], sbuf)
        pltpu.sync_copy(dst_hbm.at[pl.ds(c * CH, CH)], dbuf)

        def vec(v, cc):
            s = sbuf[pl.ds(v * 16, 16)]
            d = dbuf[pl.ds(v * 16, 16)]
            g = lax.shift_right_logical(s * 5243, 19)
            srem = s - g * NPG
            trem = d - g * NPG
            lg = g - gbase
            own = (lg >= 0) & (lg < GPT)
            base = lg * (NP * NP)
            idx1 = base + srem * NP + trem
            idx2 = base + trem * NP + srem
            zero = jnp.zeros((16,), jnp.int32)
            idx1 = jnp.where(own, idx1, zero)
            idx2 = jnp.where(own, idx2, zero)
            val = jnp.where(own, ones16, 0.0)
            plsc.addupdate_scatter(acc, [idx1], val)
            plsc.addupdate_scatter(acc, [idx2], val)
            return cc

        lax.fori_loop(0, CH // 16, vec, 0)
        return carry

    lax.fori_loop(0, E // CH, chunk, 0)

    @pl.when(wid < (G + GPT - 1) // GPT)
    def _():
        pltpu.sync_copy(acc, out_hbm.at[pl.ds(wid * (ROWS * NP), ROWS * NP)])


def _adj_sc(edge_index):
    ei = edge_index.astype(jnp.int32)
    src, dst = ei[0], ei[1]
    mesh = plsc.VectorSubcoreMesh(core_axis_name="c", subcore_axis_name="s")
    k = functools.partial(
        pl.kernel,
        mesh=mesh,
        out_type=jax.ShapeDtypeStruct((G * NP * NP,), jnp.float32),
        scratch_types=[pltpu.VMEM((ROWS * NP,), jnp.float32),
                       pltpu.VMEM((CH,), jnp.int32),
                       pltpu.VMEM((CH,), jnp.int32)],
        compiler_params=pltpu.CompilerParams(needs_layout_passes=False),
    )(_adj_sc_body)
    return k(src, dst).reshape(G, NP, NP)


def _dense_adj_pad(edge_index):
    return _adj_sc(edge_index)


def kernel(x, edge_index, batch, W1, b1, W2, b2, W3, b3):
    A0p = _dense_adj_pad(edge_index)
    X_pad = jnp.pad(x.reshape(G, NPG, D), ((0, 0), (0, NP - NPG), (0, 0)))

    X1p, X2p, X3p, summary = _megakernel(A0p, X_pad, W1, b1, W2, b2, W3, b3)

    xs0 = X1p[:, :NPG, :].reshape(-1, NHID)
    xs2 = X2p[:, :K1, :].reshape(-1, NHID)
    xs4 = X3p[:, :K2, :].reshape(-1, NHID)
    b0 = batch
    b2_ids = jnp.repeat(jnp.arange(G, dtype=jnp.int32), K1)
    b4_ids = jnp.repeat(jnp.arange(G, dtype=jnp.int32), K2)
    return (summary.reshape(G, 2 * NHID), xs0, xs2, xs4, b0, b2_ids, b4_ids)


# speedup vs baseline: 1.0814x; 1.0814x over previous
import functools
import jax, jax.numpy as jnp
from jax import lax
from jax.experimental import pallas as pl
from jax.experimental.pallas import tpu as pltpu
from jax.experimental.pallas import tpu_sc as plsc

G = 100
NPG = 100
NP = 128
N = G * NPG
D = 128
NHID = 128
K1, K1P = 50, 64
K2, K2P = 25, 32


def _eye(n):
    r = jax.lax.broadcasted_iota(jnp.int32, (n, n), 0)
    c = jax.lax.broadcasted_iota(jnp.int32, (n, n), 1)
    return jnp.where(r == c, 1.0, 0.0).astype(jnp.float32)


def _gcn_conv(A, X, W, brow, n, nreal):
    Ah = A + _eye(n)
    dc = jnp.sum(Ah, axis=1, keepdims=True)
    dr = jnp.sum(Ah, axis=0, keepdims=True)
    disc = 1.0 / jnp.sqrt(jnp.clip(dc, 1e-6))
    disr = 1.0 / jnp.sqrt(jnp.clip(dr, 1e-6))
    An = (jnp.broadcast_to(disc, (n, n)) * Ah) * jnp.broadcast_to(disr, (n, n))
    M = jax.lax.dot(An, X, preferred_element_type=jnp.float32)
    Y = jnp.maximum(jax.lax.dot(M, W, preferred_element_type=jnp.float32)
                    + jnp.broadcast_to(brow, (n, NHID)), 0.0)
    ri = jax.lax.broadcasted_iota(jnp.int32, (n, NHID), 0)
    return jnp.where(ri < nreal, Y, 0.0)


def _score_row(A, X, n):
    degc = jnp.clip(jnp.sum(A, axis=1, keepdims=True), 1.0)
    agg = jax.lax.dot(A, X, preferred_element_type=jnp.float32) \
        / jnp.broadcast_to(degc, (n, NHID))
    sc = jnp.sum(jnp.abs(X - agg), axis=1, keepdims=True)
    scT = jnp.transpose(jnp.broadcast_to(sc, (n, NHID)))
    return scT[0:1, :]


def _topk_ranks(scores, n, nreal, k):
    col = jax.lax.broadcasted_iota(jnp.int32, (G, n), 1)
    sc = jnp.where(col < nreal, scores, -1.0)
    rank0 = jnp.full((G, n), 999.0, dtype=jnp.float32)

    def body(j, carry):
        sc, rank = carry
        mx = jnp.max(sc, axis=1, keepdims=True)
        cand = sc == mx
        am = jnp.min(jnp.where(cand, col, n), axis=1, keepdims=True)
        oh = col == am
        rank = jnp.where(oh, j.astype(jnp.float32), rank)
        return jnp.where(oh, -2.0, sc), rank

    _, rank = jax.lax.fori_loop(0, k, body, (sc, rank0))
    return rank


def _P_from_rank(rankrow, kp, n):
    rk = jnp.broadcast_to(rankrow, (kp, n)).astype(jnp.int32)
    rowi = jax.lax.broadcasted_iota(jnp.int32, (kp, n), 0)
    return jnp.where(rk == rowi, 1.0, 0.0).astype(jnp.float32)


def _mega_body(A0_ref, X0_ref, W1_ref, b1_ref, W2_ref, b2_ref, W3_ref, b3_ref,
               X1_ref, X2_ref, X3_ref, sum_ref,
               sc1_ref, rk1_ref, Xp1_ref, A1_ref, sc2_ref, rk2_ref,
               Xp2_ref, A2_ref):
    W1, b1 = W1_ref[...], b1_ref[...]
    W2, b2 = W2_ref[...], b2_ref[...]
    W3, b3 = W3_ref[...], b3_ref[...]
    HI = jax.lax.Precision.HIGHEST

    def stage_a(g, c):
        A = A0_ref[g]
        X1 = _gcn_conv(A, X0_ref[g], W1, b1, NP, NPG)
        X1_ref[g] = X1
        sc1_ref[g] = _score_row(A, X1, NP)
        return c

    jax.lax.fori_loop(0, G, stage_a, 0)
    rk1_ref[...] = _topk_ranks(sc1_ref[...].reshape(G, NP), NP, NPG, K1) \
        .reshape(G, 1, NP)

    def stage_b(g, c):
        P = _P_from_rank(rk1_ref[g], K1P, NP)
        A = A0_ref[g]
        Xp = jax.lax.dot(P, X1_ref[g], precision=HI,
                         preferred_element_type=jnp.float32)
        Xp1_ref[g] = Xp
        Ar = jax.lax.dot(P, A, preferred_element_type=jnp.float32)
        A1 = jax.lax.dot_general(Ar, P, (((1,), (1,)), ((), ())),
                                 preferred_element_type=jnp.float32)
        A1_ref[g] = A1
        X2 = _gcn_conv(A1, Xp, W2, b2, K1P, K1)
        X2_ref[g] = X2
        sc2_ref[g] = _score_row(A1, X2, K1P)
        return c

    jax.lax.fori_loop(0, G, stage_b, 0)
    rk2_ref[...] = _topk_ranks(sc2_ref[...].reshape(G, K1P), K1P, K1, K2) \
        .reshape(G, 1, K1P)

    def stage_c(g, c):
        P = _P_from_rank(rk2_ref[g], K2P, K1P)
        Xp = jax.lax.dot(P, X2_ref[g], precision=HI,
                         preferred_element_type=jnp.float32)
        Xp2_ref[g] = Xp
        Ar = jax.lax.dot(P, A1_ref[g], preferred_element_type=jnp.float32)
        A2 = jax.lax.dot_general(Ar, P, (((1,), (1,)), ((), ())),
                                 preferred_element_type=jnp.float32)
        A2_ref[g] = A2
        X3 = _gcn_conv(A2, Xp, W3, b3, K2P, K2)
        X3_ref[g] = X3

        Xp1 = Xp1_ref[g]
        mx1 = jnp.max(Xp1, axis=0, keepdims=True)
        mn1 = jnp.sum(Xp1, axis=0, keepdims=True) / float(K1)
        mx2 = jnp.max(Xp, axis=0, keepdims=True)
        mn2 = jnp.sum(Xp, axis=0, keepdims=True) / float(K2)
        mx3 = jnp.max(X3, axis=0, keepdims=True)
        mn3 = jnp.sum(X3, axis=0, keepdims=True) / float(K2)
        r = jnp.maximum
        smax = r(mx1, 0.) + r(mx2, 0.) + r(mx3, 0.)
        smean = r(mn1, 0.) + r(mn2, 0.) + r(mn3, 0.)
        sum_ref[g] = jnp.concatenate([smax, smean], axis=1)
        return c

    jax.lax.fori_loop(0, G, stage_c, 0)


def _megakernel(A0p, X0p, W1, b1, W2, b2, W3, b3):
    f32 = jnp.float32
    return pl.pallas_call(
        _mega_body,
        out_shape=(jax.ShapeDtypeStruct((G, NP, NHID), f32),
                   jax.ShapeDtypeStruct((G, K1P, NHID), f32),
                   jax.ShapeDtypeStruct((G, K2P, NHID), f32),
                   jax.ShapeDtypeStruct((G, 1, 2 * NHID), f32)),
        scratch_shapes=[pltpu.VMEM((G, 1, NP), f32),
                        pltpu.VMEM((G, 1, NP), f32),
                        pltpu.VMEM((G, K1P, NHID), f32),
                        pltpu.VMEM((G, K1P, K1P), f32),
                        pltpu.VMEM((G, 1, K1P), f32),
                        pltpu.VMEM((G, 1, K1P), f32),
                        pltpu.VMEM((G, K2P, NHID), f32),
                        pltpu.VMEM((G, K2P, K2P), f32)],
    )(A0p, X0p, W1, b1.reshape(1, NHID), W2, b2.reshape(1, NHID),
      W3, b3.reshape(1, NHID))


E = 320000
GPT = 4
CH = 8000
ROWS = GPT * NP


def _adj_sc_body(src_hbm, dst_hbm, out_hbm, acc, sb0, db0, sb1, db1,
                 sem0, sem1):
    wid = lax.axis_index("s") * 2 + lax.axis_index("c")
    zero16 = jnp.zeros((16,), jnp.float32)
    ones16 = jnp.ones((16,), jnp.float32)
    zidx16 = jnp.zeros((16,), jnp.int32)

    def zbody(i, c):
        for u in range(8):
            acc[pl.ds((i * 8 + u) * 16, 16)] = zero16
        return c

    lax.fori_loop(0, ROWS * NP // 128, zbody, 0)

    gbase = wid * GPT

    def process(sbuf, dbuf):
        def vec(v, cc):
            for u in range(4):
                s = sbuf[pl.ds((v * 4 + u) * 16, 16)]
                d = dbuf[pl.ds((v * 4 + u) * 16, 16)]
                g = lax.shift_right_logical(s * 5243, 19)
                srem = s - g * NPG
                trem = d - g * NPG
                lg = g - gbase
                own = (lg >= 0) & (lg < GPT)
                base = lg * (NP * NP)
                idx1 = jnp.where(own, base + srem * NP + trem, zidx16)
                idx2 = jnp.where(own, base + trem * NP + srem, zidx16)
                val = jnp.where(own, ones16, 0.0)
                plsc.addupdate_scatter(acc, [idx1], val)
                plsc.addupdate_scatter(acc, [idx2], val)
            return cc

        lax.fori_loop(0, CH // 64, vec, 0)

    def start(c, sbuf, dbuf):
        cs = pltpu.make_async_copy(src_hbm.at[pl.ds(c * CH, CH)], sbuf, sem0)
        cd = pltpu.make_async_copy(dst_hbm.at[pl.ds(c * CH, CH)], dbuf, sem1)
        cs.start()
        cd.start()
        return cs, cd

    def wait(sbuf, dbuf):
        pltpu.make_async_copy(src_hbm.at[pl.ds(0, CH)], sbuf, sem0).wait()
        pltpu.make_async_copy(dst_hbm.at[pl.ds(0, CH)], dbuf, sem1).wait()

    start(0, sb0, db0)

    def pair(i, carry):
        c0 = i * 2
        wait(sb0, db0)
        start(c0 + 1, sb1, db1)
        process(sb0, db0)
        wait(sb1, db1)

        @pl.when(c0 + 2 < E // CH)
        def _():
            start(c0 + 2, sb0, db0)

        process(sb1, db1)
        return carry

    lax.fori_loop(0, E // CH // 2, pair, 0)

    @pl.when(wid < (G + GPT - 1) // GPT)
    def _():
        pltpu.sync_copy(acc, out_hbm.at[pl.ds(wid * (ROWS * NP), ROWS * NP)])


def _adj_sc(edge_index):
    ei = edge_index.astype(jnp.int32)
    src, dst = ei[0], ei[1]
    mesh = plsc.VectorSubcoreMesh(core_axis_name="c", subcore_axis_name="s")
    k = functools.partial(
        pl.kernel,
        mesh=mesh,
        out_type=jax.ShapeDtypeStruct((G * NP * NP,), jnp.float32),
        scratch_types=[pltpu.VMEM((ROWS * NP,), jnp.float32),
                       pltpu.VMEM((CH,), jnp.int32),
                       pltpu.VMEM((CH,), jnp.int32),
                       pltpu.VMEM((CH,), jnp.int32),
                       pltpu.VMEM((CH,), jnp.int32),
                       pltpu.SemaphoreType.DMA,
                       pltpu.SemaphoreType.DMA],
        compiler_params=pltpu.CompilerParams(needs_layout_passes=False),
    )(_adj_sc_body)
    return k(src, dst).reshape(G, NP, NP)


def _dense_adj_pad(edge_index):
    return _adj_sc(edge_index)


def kernel(x, edge_index, batch, W1, b1, W2, b2, W3, b3):
    A0p = _dense_adj_pad(edge_index)
    X_pad = jnp.pad(x.reshape(G, NPG, D), ((0, 0), (0, NP - NPG), (0, 0)))

    X1p, X2p, X3p, summary = _megakernel(A0p, X_pad, W1, b1, W2, b2, W3, b3)

    xs0 = X1p[:, :NPG, :].reshape(-1, NHID)
    xs2 = X2p[:, :K1, :].reshape(-1, NHID)
    xs4 = X3p[:, :K2, :].reshape(-1, NHID)
    b0 = batch
    b2_ids = jnp.repeat(jnp.arange(G, dtype=jnp.int32), K1)
    b4_ids = jnp.repeat(jnp.arange(G, dtype=jnp.int32), K2)
    return (summary.reshape(G, 2 * NHID), xs0, xs2, xs4, b0, b2_ids, b4_ids)


# speedup vs baseline: 2.5253x; 2.3352x over previous
import functools
import jax, jax.numpy as jnp
from jax import lax
from jax.experimental import pallas as pl
from jax.experimental.pallas import tpu as pltpu
from jax.experimental.pallas import tpu_sc as plsc

G = 100
NPG = 100
NP = 128
N = G * NPG
D = 128
NHID = 128
K1, K1P = 50, 64
K2, K2P = 25, 32


def _eye(n):
    r = jax.lax.broadcasted_iota(jnp.int32, (n, n), 0)
    c = jax.lax.broadcasted_iota(jnp.int32, (n, n), 1)
    return jnp.where(r == c, 1.0, 0.0).astype(jnp.float32)


def _gcn_conv(A, X, W, brow, n, nreal):
    Ah = A + _eye(n)
    dc = jnp.sum(Ah, axis=1, keepdims=True)
    dr = jnp.sum(Ah, axis=0, keepdims=True)
    disc = 1.0 / jnp.sqrt(jnp.clip(dc, 1e-6))
    disr = 1.0 / jnp.sqrt(jnp.clip(dr, 1e-6))
    An = (jnp.broadcast_to(disc, (n, n)) * Ah) * jnp.broadcast_to(disr, (n, n))
    M = jax.lax.dot(An, X, preferred_element_type=jnp.float32)
    Y = jnp.maximum(jax.lax.dot(M, W, preferred_element_type=jnp.float32)
                    + jnp.broadcast_to(brow, (n, NHID)), 0.0)
    ri = jax.lax.broadcasted_iota(jnp.int32, (n, NHID), 0)
    return jnp.where(ri < nreal, Y, 0.0)


def _score_row(A, X, n):
    degc = jnp.clip(jnp.sum(A, axis=1, keepdims=True), 1.0)
    agg = jax.lax.dot(A, X, preferred_element_type=jnp.float32) \
        / jnp.broadcast_to(degc, (n, NHID))
    sc = jnp.sum(jnp.abs(X - agg), axis=1, keepdims=True)
    scT = jnp.transpose(jnp.broadcast_to(sc, (n, NHID)))
    return scT[0:1, :]


def _topk_ranks(scores, n, nreal, k):
    col = jax.lax.broadcasted_iota(jnp.int32, (G, n), 1)
    sc = jnp.where(col < nreal, scores, -1.0)
    rank0 = jnp.full((G, n), 999.0, dtype=jnp.float32)

    def body(j, carry):
        sc, rank = carry
        mx = jnp.max(sc, axis=1, keepdims=True)
        cand = sc == mx
        am = jnp.min(jnp.where(cand, col, n), axis=1, keepdims=True)
        oh = col == am
        rank = jnp.where(oh, j.astype(jnp.float32), rank)
        return jnp.where(oh, -2.0, sc), rank

    _, rank = jax.lax.fori_loop(0, k, body, (sc, rank0))
    return rank


def _P_from_rank(rankrow, kp, n):
    rk = jnp.broadcast_to(rankrow, (kp, n)).astype(jnp.int32)
    rowi = jax.lax.broadcasted_iota(jnp.int32, (kp, n), 0)
    return jnp.where(rk == rowi, 1.0, 0.0).astype(jnp.float32)


def _mega_body(A0_ref, X0_ref, W1_ref, b1_ref, W2_ref, b2_ref, W3_ref, b3_ref,
               X1_ref, X2_ref, X3_ref, sum_ref,
               sc1_ref, rk1_ref, Xp1_ref, A1_ref, sc2_ref, rk2_ref,
               Xp2_ref, A2_ref):
    W1, b1 = W1_ref[...], b1_ref[...]
    W2, b2 = W2_ref[...], b2_ref[...]
    W3, b3 = W3_ref[...], b3_ref[...]
    HI = jax.lax.Precision.HIGHEST

    def stage_a(g, c):
        A = A0_ref[g]
        X1 = _gcn_conv(A, X0_ref[g], W1, b1, NP, NPG)
        X1_ref[g] = X1
        sc1_ref[g] = _score_row(A, X1, NP)
        return c

    jax.lax.fori_loop(0, G, stage_a, 0)
    rk1_ref[...] = _topk_ranks(sc1_ref[...].reshape(G, NP), NP, NPG, K1) \
        .reshape(G, 1, NP)

    def stage_b(g, c):
        P = _P_from_rank(rk1_ref[g], K1P, NP)
        A = A0_ref[g]
        Xp = jax.lax.dot(P, X1_ref[g], precision=HI,
                         preferred_element_type=jnp.float32)
        Xp1_ref[g] = Xp
        Ar = jax.lax.dot(P, A, preferred_element_type=jnp.float32)
        A1 = jax.lax.dot_general(Ar, P, (((1,), (1,)), ((), ())),
                                 preferred_element_type=jnp.float32)
        A1_ref[g] = A1
        X2 = _gcn_conv(A1, Xp, W2, b2, K1P, K1)
        X2_ref[g] = X2
        sc2_ref[g] = _score_row(A1, X2, K1P)
        return c

    jax.lax.fori_loop(0, G, stage_b, 0)
    rk2_ref[...] = _topk_ranks(sc2_ref[...].reshape(G, K1P), K1P, K1, K2) \
        .reshape(G, 1, K1P)

    def stage_c(g, c):
        P = _P_from_rank(rk2_ref[g], K2P, K1P)
        Xp = jax.lax.dot(P, X2_ref[g], precision=HI,
                         preferred_element_type=jnp.float32)
        Xp2_ref[g] = Xp
        Ar = jax.lax.dot(P, A1_ref[g], preferred_element_type=jnp.float32)
        A2 = jax.lax.dot_general(Ar, P, (((1,), (1,)), ((), ())),
                                 preferred_element_type=jnp.float32)
        A2_ref[g] = A2
        X3 = _gcn_conv(A2, Xp, W3, b3, K2P, K2)
        X3_ref[g] = X3

        Xp1 = Xp1_ref[g]
        mx1 = jnp.max(Xp1, axis=0, keepdims=True)
        mn1 = jnp.sum(Xp1, axis=0, keepdims=True) / float(K1)
        mx2 = jnp.max(Xp, axis=0, keepdims=True)
        mn2 = jnp.sum(Xp, axis=0, keepdims=True) / float(K2)
        mx3 = jnp.max(X3, axis=0, keepdims=True)
        mn3 = jnp.sum(X3, axis=0, keepdims=True) / float(K2)
        r = jnp.maximum
        smax = r(mx1, 0.) + r(mx2, 0.) + r(mx3, 0.)
        smean = r(mn1, 0.) + r(mn2, 0.) + r(mn3, 0.)
        sum_ref[g] = jnp.concatenate([smax, smean], axis=1)
        return c

    jax.lax.fori_loop(0, G, stage_c, 0)


def _megakernel(A0p, X0p, W1, b1, W2, b2, W3, b3):
    f32 = jnp.float32
    return pl.pallas_call(
        _mega_body,
        out_shape=(jax.ShapeDtypeStruct((G, NP, NHID), f32),
                   jax.ShapeDtypeStruct((G, K1P, NHID), f32),
                   jax.ShapeDtypeStruct((G, K2P, NHID), f32),
                   jax.ShapeDtypeStruct((G, 1, 2 * NHID), f32)),
        scratch_shapes=[pltpu.VMEM((G, 1, NP), f32),
                        pltpu.VMEM((G, 1, NP), f32),
                        pltpu.VMEM((G, K1P, NHID), f32),
                        pltpu.VMEM((G, K1P, K1P), f32),
                        pltpu.VMEM((G, 1, K1P), f32),
                        pltpu.VMEM((G, 1, K1P), f32),
                        pltpu.VMEM((G, K2P, NHID), f32),
                        pltpu.VMEM((G, K2P, K2P), f32)],
    )(A0p, X0p, W1, b1.reshape(1, NHID), W2, b2.reshape(1, NHID),
      W3, b3.reshape(1, NHID))


E = 320000
GPT = 4
CH = 8000
ROWS = GPT * NP


def _adj_sc_body(src_hbm, dst_hbm, out_hbm, acc, sb0, db0, sb1, db1,
                 sem0, sem1):
    wid = lax.axis_index("s") * 2 + lax.axis_index("c")
    zero16 = jnp.zeros((16,), jnp.float32)
    ones16 = jnp.ones((16,), jnp.float32)
    zidx16 = jnp.zeros((16,), jnp.int32)

    def zbody(i, c):
        for u in range(8):
            acc[pl.ds((i * 8 + u) * 16, 16)] = zero16
        return c

    lax.fori_loop(0, ROWS * NP // 128, zbody, 0)

    gbase = wid * GPT

    def process(sbuf, dbuf):
        def vec(v, cc):
            for u in range(4):
                s = sbuf[pl.ds((v * 4 + u) * 16, 16)]
                d = dbuf[pl.ds((v * 4 + u) * 16, 16)]
                g = lax.shift_right_logical(s * 5243, 19)
                srem = s - g * NPG
                trem = d - g * NPG
                lg = g - gbase
                own = (lg >= 0) & (lg < GPT)
                base = lg * (NP * NP)
                idx1 = base + srem * NP + trem
                idx2 = base + trem * NP + srem
                plsc.addupdate_scatter(acc, [idx1], ones16, mask=own)
                plsc.addupdate_scatter(acc, [idx2], ones16, mask=own)
            return cc

        lax.fori_loop(0, CH // 64, vec, 0)

    def start(c, sbuf, dbuf):
        cs = pltpu.make_async_copy(src_hbm.at[pl.ds(c * CH, CH)], sbuf, sem0)
        cd = pltpu.make_async_copy(dst_hbm.at[pl.ds(c * CH, CH)], dbuf, sem1)
        cs.start()
        cd.start()
        return cs, cd

    def wait(sbuf, dbuf):
        pltpu.make_async_copy(src_hbm.at[pl.ds(0, CH)], sbuf, sem0).wait()
        pltpu.make_async_copy(dst_hbm.at[pl.ds(0, CH)], dbuf, sem1).wait()

    start(0, sb0, db0)

    def pair(i, carry):
        c0 = i * 2
        wait(sb0, db0)
        start(c0 + 1, sb1, db1)
        process(sb0, db0)
        wait(sb1, db1)

        @pl.when(c0 + 2 < E // CH)
        def _():
            start(c0 + 2, sb0, db0)

        process(sb1, db1)
        return carry

    lax.fori_loop(0, E // CH // 2, pair, 0)

    @pl.when(wid < (G + GPT - 1) // GPT)
    def _():
        pltpu.sync_copy(acc, out_hbm.at[pl.ds(wid * (ROWS * NP), ROWS * NP)])


def _adj_sc(edge_index):
    ei = edge_index.astype(jnp.int32)
    src, dst = ei[0], ei[1]
    mesh = plsc.VectorSubcoreMesh(core_axis_name="c", subcore_axis_name="s")
    k = functools.partial(
        pl.kernel,
        mesh=mesh,
        out_type=jax.ShapeDtypeStruct((G * NP * NP,), jnp.float32),
        scratch_types=[pltpu.VMEM((ROWS * NP,), jnp.float32),
                       pltpu.VMEM((CH,), jnp.int32),
                       pltpu.VMEM((CH,), jnp.int32),
                       pltpu.VMEM((CH,), jnp.int32),
                       pltpu.VMEM((CH,), jnp.int32),
                       pltpu.SemaphoreType.DMA,
                       pltpu.SemaphoreType.DMA],
        compiler_params=pltpu.CompilerParams(needs_layout_passes=False),
    )(_adj_sc_body)
    return k(src, dst).reshape(G, NP, NP)


def _dense_adj_pad(edge_index):
    return _adj_sc(edge_index)


def kernel(x, edge_index, batch, W1, b1, W2, b2, W3, b3):
    A0p = _dense_adj_pad(edge_index)
    X_pad = jnp.pad(x.reshape(G, NPG, D), ((0, 0), (0, NP - NPG), (0, 0)))

    X1p, X2p, X3p, summary = _megakernel(A0p, X_pad, W1, b1, W2, b2, W3, b3)

    xs0 = X1p[:, :NPG, :].reshape(-1, NHID)
    xs2 = X2p[:, :K1, :].reshape(-1, NHID)
    xs4 = X3p[:, :K2, :].reshape(-1, NHID)
    b0 = batch
    b2_ids = jnp.repeat(jnp.arange(G, dtype=jnp.int32), K1)
    b4_ids = jnp.repeat(jnp.arange(G, dtype=jnp.int32), K2)
    return (summary.reshape(G, 2 * NHID), xs0, xs2, xs4, b0, b2_ids, b4_ids)


# speedup vs baseline: 3.1176x; 1.2346x over previous
import functools
import jax, jax.numpy as jnp
from jax import lax
from jax.experimental import pallas as pl
from jax.experimental.pallas import tpu as pltpu
from jax.experimental.pallas import tpu_sc as plsc

G = 100
NPG = 100
NP = 128
N = G * NPG
D = 128
NHID = 128
K1, K1P = 50, 64
K2, K2P = 25, 32


def _eye(n):
    r = jax.lax.broadcasted_iota(jnp.int32, (n, n), 0)
    c = jax.lax.broadcasted_iota(jnp.int32, (n, n), 1)
    return jnp.where(r == c, 1.0, 0.0).astype(jnp.float32)


def _gcn_conv(A, X, W, brow, n, nreal):
    Ah = A + _eye(n)
    dc = jnp.sum(Ah, axis=1, keepdims=True)
    dr = jnp.sum(Ah, axis=0, keepdims=True)
    disc = 1.0 / jnp.sqrt(jnp.clip(dc, 1e-6))
    disr = 1.0 / jnp.sqrt(jnp.clip(dr, 1e-6))
    An = (jnp.broadcast_to(disc, (n, n)) * Ah) * jnp.broadcast_to(disr, (n, n))
    M = jax.lax.dot(An, X, preferred_element_type=jnp.float32)
    Y = jnp.maximum(jax.lax.dot(M, W, preferred_element_type=jnp.float32)
                    + jnp.broadcast_to(brow, (n, NHID)), 0.0)
    ri = jax.lax.broadcasted_iota(jnp.int32, (n, NHID), 0)
    return jnp.where(ri < nreal, Y, 0.0)


def _score_row(A, X, n):
    degc = jnp.clip(jnp.sum(A, axis=1, keepdims=True), 1.0)
    agg = jax.lax.dot(A, X, preferred_element_type=jnp.float32) \
        / jnp.broadcast_to(degc, (n, NHID))
    sc = jnp.sum(jnp.abs(X - agg), axis=1, keepdims=True)
    scT = jnp.transpose(jnp.broadcast_to(sc, (n, NHID)))
    return scT[0:1, :]


def _topk_ranks(scores, n, nreal, k):
    col = jax.lax.broadcasted_iota(jnp.int32, (G, n), 1)
    sc = jnp.where(col < nreal, scores, -1.0)
    rank0 = jnp.full((G, n), 999.0, dtype=jnp.float32)

    def body(j, carry):
        sc, rank = carry
        mx = jnp.max(sc, axis=1, keepdims=True)
        cand = sc == mx
        am = jnp.min(jnp.where(cand, col, n), axis=1, keepdims=True)
        oh = col == am
        rank = jnp.where(oh, j.astype(jnp.float32), rank)
        return jnp.where(oh, -2.0, sc), rank

    _, rank = jax.lax.fori_loop(0, k, body, (sc, rank0))
    return rank


def _P_from_rank(rankrow, kp, n):
    rk = jnp.broadcast_to(rankrow, (kp, n)).astype(jnp.int32)
    rowi = jax.lax.broadcasted_iota(jnp.int32, (kp, n), 0)
    return jnp.where(rk == rowi, 1.0, 0.0).astype(jnp.float32)


def _mega_body(A0_ref, X0_ref, W1_ref, b1_ref, W2_ref, b2_ref, W3_ref, b3_ref,
               X1_ref, X2_ref, X3_ref, sum_ref,
               sc1_ref, rk1_ref, Xp1_ref, A1_ref, sc2_ref, rk2_ref,
               Xp2_ref, A2_ref, M1_ref, M2_ref, M3_ref):
    W1, b1 = W1_ref[...], b1_ref[...]
    W2, b2 = W2_ref[...], b2_ref[...]
    W3, b3 = W3_ref[...], b3_ref[...]
    HI = jax.lax.Precision.HIGHEST
    UN = 2

    def _norm_adj(A, n):
        Ah = A + _eye(n)
        dc = jnp.sum(Ah, axis=1, keepdims=True)
        dr = jnp.sum(Ah, axis=0, keepdims=True)
        disc = 1.0 / jnp.sqrt(jnp.clip(dc, 1e-6))
        disr = 1.0 / jnp.sqrt(jnp.clip(dr, 1e-6))
        return (jnp.broadcast_to(disc, (n, n)) * Ah) * jnp.broadcast_to(disr, (n, n))

    def _batched_xw(M_ref, W, b, nper, nreal, rows):
        M = M_ref[...].reshape(rows, NHID)
        Y = jnp.maximum(jax.lax.dot(M, W, preferred_element_type=jnp.float32)
                        + jnp.broadcast_to(b, (rows, NHID)), 0.0)
        ri = jax.lax.broadcasted_iota(jnp.int32, (rows, NHID), 0)
        return jnp.where(ri % nper < nreal, Y, 0.0)

    def stage_a1(i, c):
        for u in range(UN):
            g = i * UN + u
            An = _norm_adj(A0_ref[g], NP)
            M1_ref[g] = jax.lax.dot(An, X0_ref[g],
                                    preferred_element_type=jnp.float32)
        return c

    jax.lax.fori_loop(0, G // UN, stage_a1, 0)
    X1_ref[...] = _batched_xw(M1_ref, W1, b1, NP, NPG, G * NP) \
        .reshape(G, NP, NHID)

    def stage_a2(i, c):
        for u in range(UN):
            g = i * UN + u
            sc1_ref[g] = _score_row(A0_ref[g], X1_ref[g], NP)
        return c

    jax.lax.fori_loop(0, G // UN, stage_a2, 0)
    rk1_ref[...] = _topk_ranks(sc1_ref[...].reshape(G, NP), NP, NPG, K1) \
        .reshape(G, 1, NP)

    def stage_b1(i, c):
        for u in range(UN):
            g = i * UN + u
            P = _P_from_rank(rk1_ref[g], K1P, NP)
            A = A0_ref[g]
            Xp = jax.lax.dot(P, X1_ref[g], precision=HI,
                             preferred_element_type=jnp.float32)
            Xp1_ref[g] = Xp
            Ar = jax.lax.dot(P, A, preferred_element_type=jnp.float32)
            A1 = jax.lax.dot_general(Ar, P, (((1,), (1,)), ((), ())),
                                     preferred_element_type=jnp.float32)
            A1_ref[g] = A1
            An = _norm_adj(A1, K1P)
            M2_ref[g] = jax.lax.dot(An, Xp, preferred_element_type=jnp.float32)
        return c

    jax.lax.fori_loop(0, G // UN, stage_b1, 0)
    X2_ref[...] = _batched_xw(M2_ref, W2, b2, K1P, K1, G * K1P) \
        .reshape(G, K1P, NHID)

    def stage_b2(i, c):
        for u in range(UN):
            g = i * UN + u
            sc2_ref[g] = _score_row(A1_ref[g], X2_ref[g], K1P)
        return c

    jax.lax.fori_loop(0, G // UN, stage_b2, 0)
    rk2_ref[...] = _topk_ranks(sc2_ref[...].reshape(G, K1P), K1P, K1, K2) \
        .reshape(G, 1, K1P)

    def stage_c1(i, c):
        for u in range(UN):
            g = i * UN + u
            P = _P_from_rank(rk2_ref[g], K2P, K1P)
            Xp = jax.lax.dot(P, X2_ref[g], precision=HI,
                             preferred_element_type=jnp.float32)
            Xp2_ref[g] = Xp
            Ar = jax.lax.dot(P, A1_ref[g], preferred_element_type=jnp.float32)
            A2 = jax.lax.dot_general(Ar, P, (((1,), (1,)), ((), ())),
                                     preferred_element_type=jnp.float32)
            A2_ref[g] = A2
            An = _norm_adj(A2, K2P)
            M3_ref[g] = jax.lax.dot(An, Xp, preferred_element_type=jnp.float32)
        return c

    jax.lax.fori_loop(0, G // UN, stage_c1, 0)
    X3_ref[...] = _batched_xw(M3_ref, W3, b3, K2P, K2, G * K2P) \
        .reshape(G, K2P, NHID)

    def stage_c2(i, c):
        for u in range(UN):
            g = i * UN + u
            Xp1 = Xp1_ref[g]
            Xp2 = Xp2_ref[g]
            X3 = X3_ref[g]
            mx1 = jnp.max(Xp1, axis=0, keepdims=True)
            mn1 = jnp.sum(Xp1, axis=0, keepdims=True) / float(K1)
            mx2 = jnp.max(Xp2, axis=0, keepdims=True)
            mn2 = jnp.sum(Xp2, axis=0, keepdims=True) / float(K2)
            mx3 = jnp.max(X3, axis=0, keepdims=True)
            mn3 = jnp.sum(X3, axis=0, keepdims=True) / float(K2)
            r = jnp.maximum
            smax = r(mx1, 0.) + r(mx2, 0.) + r(mx3, 0.)
            smean = r(mn1, 0.) + r(mn2, 0.) + r(mn3, 0.)
            sum_ref[g] = jnp.concatenate([smax, smean], axis=1)
        return c

    jax.lax.fori_loop(0, G // UN, stage_c2, 0)


def _megakernel(A0p, X0p, W1, b1, W2, b2, W3, b3):
    f32 = jnp.float32
    return pl.pallas_call(
        _mega_body,
        out_shape=(jax.ShapeDtypeStruct((G, NP, NHID), f32),
                   jax.ShapeDtypeStruct((G, K1P, NHID), f32),
                   jax.ShapeDtypeStruct((G, K2P, NHID), f32),
                   jax.ShapeDtypeStruct((G, 1, 2 * NHID), f32)),
        scratch_shapes=[pltpu.VMEM((G, 1, NP), f32),
                        pltpu.VMEM((G, 1, NP), f32),
                        pltpu.VMEM((G, K1P, NHID), f32),
                        pltpu.VMEM((G, K1P, K1P), f32),
                        pltpu.VMEM((G, 1, K1P), f32),
                        pltpu.VMEM((G, 1, K1P), f32),
                        pltpu.VMEM((G, K2P, NHID), f32),
                        pltpu.VMEM((G, K2P, K2P), f32),
                        pltpu.VMEM((G, NP, NHID), f32),
                        pltpu.VMEM((G, K1P, NHID), f32),
                        pltpu.VMEM((G, K2P, NHID), f32)],
    )(A0p, X0p, W1, b1.reshape(1, NHID), W2, b2.reshape(1, NHID),
      W3, b3.reshape(1, NHID))


E = 320000
GPT = 4
CH = 6400
ROWS = GPT * NP


def _adj_sc_body(src_hbm, dst_hbm, out_hbm, acc, sb0, db0, sb1, db1,
                 sem0, sem1):
    wid = lax.axis_index("s") * 2 + lax.axis_index("c")
    zero16 = jnp.zeros((16,), jnp.float32)
    ones16 = jnp.ones((16,), jnp.float32)
    zidx16 = jnp.zeros((16,), jnp.int32)

    def zbody(i, c):
        for u in range(8):
            acc[pl.ds((i * 8 + u) * 16, 16)] = zero16
        return c

    lax.fori_loop(0, ROWS * NP // 128, zbody, 0)

    gbase = wid * GPT

    def process(sbuf, dbuf):
        def vec(v, cc):
            for u in range(8):
                s = sbuf[pl.ds((v * 8 + u) * 16, 16)]
                d = dbuf[pl.ds((v * 8 + u) * 16, 16)]
                g = lax.shift_right_logical(s * 5243, 19)
                srem = s - g * NPG
                trem = d - g * NPG
                lg = g - gbase
                own = (lg >= 0) & (lg < GPT)
                base = lg * (NP * NP)
                idx1 = base + srem * NP + trem
                idx2 = base + trem * NP + srem
                plsc.addupdate_scatter(acc, [idx1], ones16, mask=own)
                plsc.addupdate_scatter(acc, [idx2], ones16, mask=own)
            return cc

        lax.fori_loop(0, CH // 128, vec, 0)

    def start(c, sbuf, dbuf):
        cs = pltpu.make_async_copy(src_hbm.at[pl.ds(c * CH, CH)], sbuf, sem0)
        cd = pltpu.make_async_copy(dst_hbm.at[pl.ds(c * CH, CH)], dbuf, sem1)
        cs.start()
        cd.start()
        return cs, cd

    def wait(sbuf, dbuf):
        pltpu.make_async_copy(src_hbm.at[pl.ds(0, CH)], sbuf, sem0).wait()
        pltpu.make_async_copy(dst_hbm.at[pl.ds(0, CH)], dbuf, sem1).wait()

    start(0, sb0, db0)

    def pair(i, carry):
        c0 = i * 2
        wait(sb0, db0)
        start(c0 + 1, sb1, db1)
        process(sb0, db0)
        wait(sb1, db1)

        @pl.when(c0 + 2 < E // CH)
        def _():
            start(c0 + 2, sb0, db0)

        process(sb1, db1)
        return carry

    lax.fori_loop(0, E // CH // 2, pair, 0)

    @pl.when(wid < (G + GPT - 1) // GPT)
    def _():
        pltpu.sync_copy(acc, out_hbm.at[pl.ds(wid * (ROWS * NP), ROWS * NP)])


def _adj_sc(edge_index):
    ei = edge_index.astype(jnp.int32)
    src, dst = ei[0], ei[1]
    mesh = plsc.VectorSubcoreMesh(core_axis_name="c", subcore_axis_name="s")
    k = functools.partial(
        pl.kernel,
        mesh=mesh,
        out_type=jax.ShapeDtypeStruct((G * NP * NP,), jnp.float32),
        scratch_types=[pltpu.VMEM((ROWS * NP,), jnp.float32),
                       pltpu.VMEM((CH,), jnp.int32),
                       pltpu.VMEM((CH,), jnp.int32),
                       pltpu.VMEM((CH,), jnp.int32),
                       pltpu.VMEM((CH,), jnp.int32),
                       pltpu.SemaphoreType.DMA,
                       pltpu.SemaphoreType.DMA],
        compiler_params=pltpu.CompilerParams(needs_layout_passes=False),
    )(_adj_sc_body)
    return k(src, dst).reshape(G, NP, NP)


def _dense_adj_pad(edge_index):
    return _adj_sc(edge_index)


def kernel(x, edge_index, batch, W1, b1, W2, b2, W3, b3):
    A0p = _dense_adj_pad(edge_index)
    X_pad = jnp.pad(x.reshape(G, NPG, D), ((0, 0), (0, NP - NPG), (0, 0)))

    X1p, X2p, X3p, summary = _megakernel(A0p, X_pad, W1, b1, W2, b2, W3, b3)

    xs0 = X1p[:, :NPG, :].reshape(-1, NHID)
    xs2 = X2p[:, :K1, :].reshape(-1, NHID)
    xs4 = X3p[:, :K2, :].reshape(-1, NHID)
    b0 = batch
    b2_ids = jnp.repeat(jnp.arange(G, dtype=jnp.int32), K1)
    b4_ids = jnp.repeat(jnp.arange(G, dtype=jnp.int32), K2)
    return (summary.reshape(G, 2 * NHID), xs0, xs2, xs4, b0, b2_ids, b4_ids)


# speedup vs baseline: 4.0659x; 1.3042x over previous
import functools
import jax, jax.numpy as jnp
from jax import lax
from jax.experimental import pallas as pl
from jax.experimental.pallas import tpu as pltpu
from jax.experimental.pallas import tpu_sc as plsc

G = 100
NPG = 100
NP = 128
N = G * NPG
D = 128
NHID = 128
K1, K1P = 50, 64
K2, K2P = 25, 32


def _eye(n):
    r = jax.lax.broadcasted_iota(jnp.int32, (n, n), 0)
    c = jax.lax.broadcasted_iota(jnp.int32, (n, n), 1)
    return jnp.where(r == c, 1.0, 0.0).astype(jnp.float32)


def _gcn_conv(A, X, W, brow, n, nreal):
    Ah = A + _eye(n)
    dc = jnp.sum(Ah, axis=1, keepdims=True)
    dr = jnp.sum(Ah, axis=0, keepdims=True)
    disc = 1.0 / jnp.sqrt(jnp.clip(dc, 1e-6))
    disr = 1.0 / jnp.sqrt(jnp.clip(dr, 1e-6))
    An = (jnp.broadcast_to(disc, (n, n)) * Ah) * jnp.broadcast_to(disr, (n, n))
    M = jax.lax.dot(An, X, preferred_element_type=jnp.float32)
    Y = jnp.maximum(jax.lax.dot(M, W, preferred_element_type=jnp.float32)
                    + jnp.broadcast_to(brow, (n, NHID)), 0.0)
    ri = jax.lax.broadcasted_iota(jnp.int32, (n, NHID), 0)
    return jnp.where(ri < nreal, Y, 0.0)


def _score_row(A, X, n):
    degc = jnp.clip(jnp.sum(A, axis=1, keepdims=True), 1.0)
    agg = jax.lax.dot(A, X, preferred_element_type=jnp.float32) \
        / jnp.broadcast_to(degc, (n, NHID))
    sc = jnp.sum(jnp.abs(X - agg), axis=1, keepdims=True)
    scT = jnp.transpose(jnp.broadcast_to(sc, (n, NHID)))
    return scT[0:1, :]


def _topk_ranks(scores, n, nreal, k):
    col = jax.lax.broadcasted_iota(jnp.int32, (G, n), 1)
    sc = jnp.where(col < nreal, scores, -1.0)
    rank0 = jnp.full((G, n), 999.0, dtype=jnp.float32)

    def body(j, carry):
        sc, rank = carry
        mx = jnp.max(sc, axis=1, keepdims=True)
        cand = sc == mx
        am = jnp.min(jnp.where(cand, col, n), axis=1, keepdims=True)
        oh = col == am
        rank = jnp.where(oh, j.astype(jnp.float32), rank)
        return jnp.where(oh, -2.0, sc), rank

    _, rank = jax.lax.fori_loop(0, k, body, (sc, rank0))
    return rank


def _P_from_rank(rankrow, kp, n):
    rk = jnp.broadcast_to(rankrow, (kp, n)).astype(jnp.int32)
    rowi = jax.lax.broadcasted_iota(jnp.int32, (kp, n), 0)
    return jnp.where(rk == rowi, 1.0, 0.0).astype(jnp.float32)


def _mega_body(A0a_ref, A0b_ref, X0_ref, W1_ref, b1_ref, W2_ref, b2_ref, W3_ref, b3_ref,
               X1_ref, X2_ref, X3_ref, sum_ref,
               sc1_ref, rk1_ref, Xp1_ref, A1_ref, sc2_ref, rk2_ref,
               Xp2_ref, A2_ref, M1_ref, M2_ref, M3_ref, A0_ref):
    W1, b1 = W1_ref[...], b1_ref[...]
    W2, b2 = W2_ref[...], b2_ref[...]
    W3, b3 = W3_ref[...], b3_ref[...]
    HI = jax.lax.Precision.HIGHEST
    UN = 2
    zpad = jnp.zeros((NP - NPG, NP), jnp.float32)

    def stage_a0(i, c):
        for u in range(UN):
            g = i * UN + u
            A0_ref[g] = jnp.concatenate([A0a_ref[g] + A0b_ref[g], zpad], axis=0)
        return c

    jax.lax.fori_loop(0, G // UN, stage_a0, 0)

    def _norm_adj(A, n):
        Ah = A + _eye(n)
        dc = jnp.sum(Ah, axis=1, keepdims=True)
        dr = jnp.sum(Ah, axis=0, keepdims=True)
        disc = 1.0 / jnp.sqrt(jnp.clip(dc, 1e-6))
        disr = 1.0 / jnp.sqrt(jnp.clip(dr, 1e-6))
        return (jnp.broadcast_to(disc, (n, n)) * Ah) * jnp.broadcast_to(disr, (n, n))

    def _batched_xw(M_ref, W, b, nper, nreal, rows):
        M = M_ref[...].reshape(rows, NHID)
        Y = jnp.maximum(jax.lax.dot(M, W, preferred_element_type=jnp.float32)
                        + jnp.broadcast_to(b, (rows, NHID)), 0.0)
        ri = jax.lax.broadcasted_iota(jnp.int32, (rows, NHID), 0)
        return jnp.where(ri % nper < nreal, Y, 0.0)

    def stage_a1(i, c):
        for u in range(UN):
            g = i * UN + u
            An = _norm_adj(A0_ref[g], NP)
            M1_ref[g] = jax.lax.dot(An, X0_ref[g],
                                    preferred_element_type=jnp.float32)
        return c

    jax.lax.fori_loop(0, G // UN, stage_a1, 0)
    X1_ref[...] = _batched_xw(M1_ref, W1, b1, NP, NPG, G * NP) \
        .reshape(G, NP, NHID)

    def stage_a2(i, c):
        for u in range(UN):
            g = i * UN + u
            sc1_ref[g] = _score_row(A0_ref[g], X1_ref[g], NP)
        return c

    jax.lax.fori_loop(0, G // UN, stage_a2, 0)
    rk1_ref[...] = _topk_ranks(sc1_ref[...].reshape(G, NP), NP, NPG, K1) \
        .reshape(G, 1, NP)

    def stage_b1(i, c):
        for u in range(UN):
            g = i * UN + u
            P = _P_from_rank(rk1_ref[g], K1P, NP)
            A = A0_ref[g]
            Xp = jax.lax.dot(P, X1_ref[g], precision=HI,
                             preferred_element_type=jnp.float32)
            Xp1_ref[g] = Xp
            Ar = jax.lax.dot(P, A, preferred_element_type=jnp.float32)
            A1 = jax.lax.dot_general(Ar, P, (((1,), (1,)), ((), ())),
                                     preferred_element_type=jnp.float32)
            A1_ref[g] = A1
            An = _norm_adj(A1, K1P)
            M2_ref[g] = jax.lax.dot(An, Xp, preferred_element_type=jnp.float32)
        return c

    jax.lax.fori_loop(0, G // UN, stage_b1, 0)
    X2_ref[...] = _batched_xw(M2_ref, W2, b2, K1P, K1, G * K1P) \
        .reshape(G, K1P, NHID)

    def stage_b2(i, c):
        for u in range(UN):
            g = i * UN + u
            sc2_ref[g] = _score_row(A1_ref[g], X2_ref[g], K1P)
        return c

    jax.lax.fori_loop(0, G // UN, stage_b2, 0)
    rk2_ref[...] = _topk_ranks(sc2_ref[...].reshape(G, K1P), K1P, K1, K2) \
        .reshape(G, 1, K1P)

    def stage_c1(i, c):
        for u in range(UN):
            g = i * UN + u
            P = _P_from_rank(rk2_ref[g], K2P, K1P)
            Xp = jax.lax.dot(P, X2_ref[g], precision=HI,
                             preferred_element_type=jnp.float32)
            Xp2_ref[g] = Xp
            Ar = jax.lax.dot(P, A1_ref[g], preferred_element_type=jnp.float32)
            A2 = jax.lax.dot_general(Ar, P, (((1,), (1,)), ((), ())),
                                     preferred_element_type=jnp.float32)
            A2_ref[g] = A2
            An = _norm_adj(A2, K2P)
            M3_ref[g] = jax.lax.dot(An, Xp, preferred_element_type=jnp.float32)
        return c

    jax.lax.fori_loop(0, G // UN, stage_c1, 0)
    X3_ref[...] = _batched_xw(M3_ref, W3, b3, K2P, K2, G * K2P) \
        .reshape(G, K2P, NHID)

    def stage_c2(i, c):
        for u in range(UN):
            g = i * UN + u
            Xp1 = Xp1_ref[g]
            Xp2 = Xp2_ref[g]
            X3 = X3_ref[g]
            mx1 = jnp.max(Xp1, axis=0, keepdims=True)
            mn1 = jnp.sum(Xp1, axis=0, keepdims=True) / float(K1)
            mx2 = jnp.max(Xp2, axis=0, keepdims=True)
            mn2 = jnp.sum(Xp2, axis=0, keepdims=True) / float(K2)
            mx3 = jnp.max(X3, axis=0, keepdims=True)
            mn3 = jnp.sum(X3, axis=0, keepdims=True) / float(K2)
            r = jnp.maximum
            smax = r(mx1, 0.) + r(mx2, 0.) + r(mx3, 0.)
            smean = r(mn1, 0.) + r(mn2, 0.) + r(mn3, 0.)
            sum_ref[g] = jnp.concatenate([smax, smean], axis=1)
        return c

    jax.lax.fori_loop(0, G // UN, stage_c2, 0)


def _megakernel(A0a, A0b, X0p, W1, b1, W2, b2, W3, b3):
    f32 = jnp.float32
    return pl.pallas_call(
        _mega_body,
        out_shape=(jax.ShapeDtypeStruct((G, NP, NHID), f32),
                   jax.ShapeDtypeStruct((G, K1P, NHID), f32),
                   jax.ShapeDtypeStruct((G, K2P, NHID), f32),
                   jax.ShapeDtypeStruct((G, 1, 2 * NHID), f32)),
        scratch_shapes=[pltpu.VMEM((G, 1, NP), f32),
                        pltpu.VMEM((G, 1, NP), f32),
                        pltpu.VMEM((G, K1P, NHID), f32),
                        pltpu.VMEM((G, K1P, K1P), f32),
                        pltpu.VMEM((G, 1, K1P), f32),
                        pltpu.VMEM((G, 1, K1P), f32),
                        pltpu.VMEM((G, K2P, NHID), f32),
                        pltpu.VMEM((G, K2P, K2P), f32),
                        pltpu.VMEM((G, NP, NHID), f32),
                        pltpu.VMEM((G, K1P, NHID), f32),
                        pltpu.VMEM((G, K2P, NHID), f32),
                        pltpu.VMEM((G, NP, NP), f32)],
    )(A0a, A0b, X0p, W1, b1.reshape(1, NHID), W2, b2.reshape(1, NHID),
      W3, b3.reshape(1, NHID))


E = 320000
GPG = 8
NGRP = 13
CH = 4000
NCHUNK = E // CH
HALF = NCHUNK // 2
AWORDS = GPG * NPG * NP
OUTW = NGRP * AWORDS


def _adj_sc_body(src_hbm, dst_hbm, out0_hbm, out1_hbm, acc, sb0, db0, sb1, db1,
                 sem0, sem1):
    wid = lax.axis_index("s") * 2 + lax.axis_index("c")
    role = wid % 2
    group = wid // 2
    zero16 = jnp.zeros((16,), jnp.float32)
    ones16 = jnp.ones((16,), jnp.float32)

    def zbody(i, c):
        for u in range(8):
            acc[pl.ds((i * 8 + u) * 16, 16)] = zero16
        return c

    lax.fori_loop(0, AWORDS // 128, zbody, 0)

    gbase = group * GPG
    cbase = role * HALF

    def process(sbuf, dbuf):
        def vec(v, cc):
            for u in range(5):
                s = sbuf[pl.ds((v * 5 + u) * 16, 16)]
                d = dbuf[pl.ds((v * 5 + u) * 16, 16)]
                g = lax.shift_right_logical(s * 5243, 19)
                srem = s - g * NPG
                trem = d - g * NPG
                lg = g - gbase
                own = (lg >= 0) & (lg < GPG)
                base = lg * (NPG * NP)
                idx1 = base + srem * NP + trem
                idx2 = base + trem * NP + srem
                plsc.addupdate_scatter(acc, [idx1], ones16, mask=own)
                plsc.addupdate_scatter(acc, [idx2], ones16, mask=own)
            return cc

        lax.fori_loop(0, CH // 80, vec, 0)

    def start(c, sbuf, dbuf):
        pltpu.make_async_copy(src_hbm.at[pl.ds(c * CH, CH)], sbuf, sem0).start()
        pltpu.make_async_copy(dst_hbm.at[pl.ds(c * CH, CH)], dbuf, sem1).start()

    def wait(sbuf, dbuf):
        pltpu.make_async_copy(src_hbm.at[pl.ds(0, CH)], sbuf, sem0).wait()
        pltpu.make_async_copy(dst_hbm.at[pl.ds(0, CH)], dbuf, sem1).wait()

    @pl.when(wid < 2 * NGRP)
    def _scan():
        start(cbase, sb0, db0)

        def pair(i, carry):
            c0 = cbase + i * 2
            wait(sb0, db0)
            start(c0 + 1, sb1, db1)
            process(sb0, db0)
            wait(sb1, db1)

            @pl.when(i + 1 < HALF // 2)
            def _():
                start(c0 + 2, sb0, db0)

            process(sb1, db1)
            return carry

        lax.fori_loop(0, HALF // 2, pair, 0)

        @pl.when(role == 0)
        def _():
            pltpu.sync_copy(acc, out0_hbm.at[pl.ds(group * AWORDS, AWORDS)])

        @pl.when(role == 1)
        def _():
            pltpu.sync_copy(acc, out1_hbm.at[pl.ds(group * AWORDS, AWORDS)])


def _adj_sc(edge_index):
    ei = edge_index.astype(jnp.int32)
    src, dst = ei[0], ei[1]
    mesh = plsc.VectorSubcoreMesh(core_axis_name="c", subcore_axis_name="s")
    k = functools.partial(
        pl.kernel,
        mesh=mesh,
        out_type=(jax.ShapeDtypeStruct((OUTW,), jnp.float32),
                  jax.ShapeDtypeStruct((OUTW,), jnp.float32)),
        scratch_types=[pltpu.VMEM((AWORDS,), jnp.float32),
                       pltpu.VMEM((CH,), jnp.int32),
                       pltpu.VMEM((CH,), jnp.int32),
                       pltpu.VMEM((CH,), jnp.int32),
                       pltpu.VMEM((CH,), jnp.int32),
                       pltpu.SemaphoreType.DMA,
                       pltpu.SemaphoreType.DMA],
        compiler_params=pltpu.CompilerParams(needs_layout_passes=False),
    )(_adj_sc_body)
    a, b = k(src, dst)
    a = a.reshape(NGRP * GPG, NPG, NP)[:G]
    b = b.reshape(NGRP * GPG, NPG, NP)[:G]
    return a, b


def kernel(x, edge_index, batch, W1, b1, W2, b2, W3, b3):
    A0a, A0b = _adj_sc(edge_index)
    X_pad = jnp.pad(x.reshape(G, NPG, D), ((0, 0), (0, NP - NPG), (0, 0)))

    X1p, X2p, X3p, summary = _megakernel(A0a, A0b, X_pad, W1, b1, W2, b2,
                                         W3, b3)

    xs0 = X1p[:, :NPG, :].reshape(-1, NHID)
    xs2 = X2p[:, :K1, :].reshape(-1, NHID)
    xs4 = X3p[:, :K2, :].reshape(-1, NHID)
    b0 = batch
    b2_ids = jnp.repeat(jnp.arange(G, dtype=jnp.int32), K1)
    b4_ids = jnp.repeat(jnp.arange(G, dtype=jnp.int32), K2)
    return (summary.reshape(G, 2 * NHID), xs0, xs2, xs4, b0, b2_ids, b4_ids)


# speedup vs baseline: 4.4874x; 1.1037x over previous
import functools
import jax, jax.numpy as jnp
from jax import lax
from jax.experimental import pallas as pl
from jax.experimental.pallas import tpu as pltpu
from jax.experimental.pallas import tpu_sc as plsc

G = 100
NPG = 100
NP = 128
N = G * NPG
D = 128
NHID = 128
K1, K1P = 50, 64
K2, K2P = 25, 32


def _eye(n):
    r = jax.lax.broadcasted_iota(jnp.int32, (n, n), 0)
    c = jax.lax.broadcasted_iota(jnp.int32, (n, n), 1)
    return jnp.where(r == c, 1.0, 0.0).astype(jnp.float32)


def _gcn_conv(A, X, W, brow, n, nreal):
    Ah = A + _eye(n)
    dc = jnp.sum(Ah, axis=1, keepdims=True)
    dr = jnp.sum(Ah, axis=0, keepdims=True)
    disc = 1.0 / jnp.sqrt(jnp.clip(dc, 1e-6))
    disr = 1.0 / jnp.sqrt(jnp.clip(dr, 1e-6))
    An = (jnp.broadcast_to(disc, (n, n)) * Ah) * jnp.broadcast_to(disr, (n, n))
    M = jax.lax.dot(An, X, preferred_element_type=jnp.float32)
    Y = jnp.maximum(jax.lax.dot(M, W, preferred_element_type=jnp.float32)
                    + jnp.broadcast_to(brow, (n, NHID)), 0.0)
    ri = jax.lax.broadcasted_iota(jnp.int32, (n, NHID), 0)
    return jnp.where(ri < nreal, Y, 0.0)


def _score_row(A, X, n):
    degc = jnp.clip(jnp.sum(A, axis=1, keepdims=True), 1.0)
    agg = jax.lax.dot(A, X, preferred_element_type=jnp.float32) \
        / jnp.broadcast_to(degc, (n, NHID))
    sc = jnp.sum(jnp.abs(X - agg), axis=1, keepdims=True)
    scT = jnp.transpose(jnp.broadcast_to(sc, (n, NHID)))
    return scT[0:1, :]


def _topk_ranks(scores, n, nreal, k):
    col = jax.lax.broadcasted_iota(jnp.int32, (G, n), 1)
    sc = jnp.where(col < nreal, scores, -1.0)
    rank0 = jnp.full((G, n), 999.0, dtype=jnp.float32)

    def body(j, carry):
        sc, rank = carry
        mx = jnp.max(sc, axis=1, keepdims=True)
        cand = sc == mx
        am = jnp.min(jnp.where(cand, col, n), axis=1, keepdims=True)
        oh = col == am
        rank = jnp.where(oh, j.astype(jnp.float32), rank)
        return jnp.where(oh, -2.0, sc), rank

    _, rank = jax.lax.fori_loop(0, k, body, (sc, rank0))
    return rank


def _P_from_rank(rankrow, kp, n):
    rk = jnp.broadcast_to(rankrow, (kp, n)).astype(jnp.int32)
    rowi = jax.lax.broadcasted_iota(jnp.int32, (kp, n), 0)
    return jnp.where(rk == rowi, 1.0, 0.0).astype(jnp.float32)


def _mega_body(A0a_ref, A0b_ref, X0_ref, W1_ref, b1_ref, W2_ref, b2_ref, W3_ref, b3_ref,
               X1_ref, X2_ref, X3_ref, sum_ref,
               sc1_ref, rk1_ref, Xp1_ref, A1_ref, sc2_ref, rk2_ref,
               Xp2_ref, A2_ref, M1_ref, M2_ref, M3_ref, A0_ref):
    W1, b1 = W1_ref[...], b1_ref[...]
    W2, b2 = W2_ref[...], b2_ref[...]
    W3, b3 = W3_ref[...], b3_ref[...]
    HI = jax.lax.Precision.HIGHEST
    UN = 4
    zpad = jnp.zeros((NP - NPG, NP), jnp.float32)

    def stage_a0(i, c):
        for u in range(UN):
            g = i * UN + u
            A0_ref[g] = jnp.concatenate([A0a_ref[g] + A0b_ref[g], zpad], axis=0)
        return c

    jax.lax.fori_loop(0, G // UN, stage_a0, 0)

    def _norm_adj(A, n):
        Ah = A + _eye(n)
        dc = jnp.sum(Ah, axis=1, keepdims=True)
        dr = jnp.sum(Ah, axis=0, keepdims=True)
        disc = 1.0 / jnp.sqrt(jnp.clip(dc, 1e-6))
        disr = 1.0 / jnp.sqrt(jnp.clip(dr, 1e-6))
        return (jnp.broadcast_to(disc, (n, n)) * Ah) * jnp.broadcast_to(disr, (n, n))

    def _batched_xw(M_ref, W, b, nper, nreal, rows):
        M = M_ref[...].reshape(rows, NHID)
        Y = jnp.maximum(jax.lax.dot(M, W, preferred_element_type=jnp.float32)
                        + jnp.broadcast_to(b, (rows, NHID)), 0.0)
        ri = jax.lax.broadcasted_iota(jnp.int32, (rows, NHID), 0)
        return jnp.where(ri % nper < nreal, Y, 0.0)

    def stage_a1(i, c):
        for u in range(UN):
            g = i * UN + u
            An = _norm_adj(A0_ref[g], NP)
            M1_ref[g] = jax.lax.dot(An, X0_ref[g],
                                    preferred_element_type=jnp.float32)
        return c

    jax.lax.fori_loop(0, G // UN, stage_a1, 0)
    X1_ref[...] = _batched_xw(M1_ref, W1, b1, NP, NPG, G * NP) \
        .reshape(G, NP, NHID)

    def stage_a2(i, c):
        for u in range(UN):
            g = i * UN + u
            sc1_ref[g] = _score_row(A0_ref[g], X1_ref[g], NP)
        return c

    jax.lax.fori_loop(0, G // UN, stage_a2, 0)
    rk1_ref[...] = _topk_ranks(sc1_ref[...].reshape(G, NP), NP, NPG, K1) \
        .reshape(G, 1, NP)

    def stage_b1(i, c):
        for u in range(UN):
            g = i * UN + u
            P = _P_from_rank(rk1_ref[g], K1P, NP)
            A = A0_ref[g]
            Xp = jax.lax.dot(P, X1_ref[g], precision=HI,
                             preferred_element_type=jnp.float32)
            Xp1_ref[g] = Xp
            Ar = jax.lax.dot(P, A, preferred_element_type=jnp.float32)
            A1 = jax.lax.dot_general(Ar, P, (((1,), (1,)), ((), ())),
                                     preferred_element_type=jnp.float32)
            A1_ref[g] = A1
            An = _norm_adj(A1, K1P)
            M2_ref[g] = jax.lax.dot(An, Xp, preferred_element_type=jnp.float32)
        return c

    jax.lax.fori_loop(0, G // UN, stage_b1, 0)
    X2_ref[...] = _batched_xw(M2_ref, W2, b2, K1P, K1, G * K1P) \
        .reshape(G, K1P, NHID)

    def stage_b2(i, c):
        for u in range(UN):
            g = i * UN + u
            sc2_ref[g] = _score_row(A1_ref[g], X2_ref[g], K1P)
        return c

    jax.lax.fori_loop(0, G // UN, stage_b2, 0)
    rk2_ref[...] = _topk_ranks(sc2_ref[...].reshape(G, K1P), K1P, K1, K2) \
        .reshape(G, 1, K1P)

    def stage_c1(i, c):
        for u in range(UN):
            g = i * UN + u
            P = _P_from_rank(rk2_ref[g], K2P, K1P)
            Xp = jax.lax.dot(P, X2_ref[g], precision=HI,
                             preferred_element_type=jnp.float32)
            Xp2_ref[g] = Xp
            Ar = jax.lax.dot(P, A1_ref[g], preferred_element_type=jnp.float32)
            A2 = jax.lax.dot_general(Ar, P, (((1,), (1,)), ((), ())),
                                     preferred_element_type=jnp.float32)
            A2_ref[g] = A2
            An = _norm_adj(A2, K2P)
            M3_ref[g] = jax.lax.dot(An, Xp, preferred_element_type=jnp.float32)
        return c

    jax.lax.fori_loop(0, G // UN, stage_c1, 0)
    X3_ref[...] = _batched_xw(M3_ref, W3, b3, K2P, K2, G * K2P) \
        .reshape(G, K2P, NHID)

    def stage_c2(i, c):
        for u in range(UN):
            g = i * UN + u
            Xp1 = Xp1_ref[g]
            Xp2 = Xp2_ref[g]
            X3 = X3_ref[g]
            mx1 = jnp.max(Xp1, axis=0, keepdims=True)
            mn1 = jnp.sum(Xp1, axis=0, keepdims=True) / float(K1)
            mx2 = jnp.max(Xp2, axis=0, keepdims=True)
            mn2 = jnp.sum(Xp2, axis=0, keepdims=True) / float(K2)
            mx3 = jnp.max(X3, axis=0, keepdims=True)
            mn3 = jnp.sum(X3, axis=0, keepdims=True) / float(K2)
            r = jnp.maximum
            smax = r(mx1, 0.) + r(mx2, 0.) + r(mx3, 0.)
            smean = r(mn1, 0.) + r(mn2, 0.) + r(mn3, 0.)
            sum_ref[g] = jnp.concatenate([smax, smean], axis=1)
        return c

    jax.lax.fori_loop(0, G // UN, stage_c2, 0)


def _megakernel(A0a, A0b, X0p, W1, b1, W2, b2, W3, b3):
    f32 = jnp.float32
    return pl.pallas_call(
        _mega_body,
        out_shape=(jax.ShapeDtypeStruct((G, NP, NHID), f32),
                   jax.ShapeDtypeStruct((G, K1P, NHID), f32),
                   jax.ShapeDtypeStruct((G, K2P, NHID), f32),
                   jax.ShapeDtypeStruct((G, 1, 2 * NHID), f32)),
        scratch_shapes=[pltpu.VMEM((G, 1, NP), f32),
                        pltpu.VMEM((G, 1, NP), f32),
                        pltpu.VMEM((G, K1P, NHID), f32),
                        pltpu.VMEM((G, K1P, K1P), f32),
                        pltpu.VMEM((G, 1, K1P), f32),
                        pltpu.VMEM((G, 1, K1P), f32),
                        pltpu.VMEM((G, K2P, NHID), f32),
                        pltpu.VMEM((G, K2P, K2P), f32),
                        pltpu.VMEM((G, NP, NHID), f32),
                        pltpu.VMEM((G, K1P, NHID), f32),
                        pltpu.VMEM((G, K2P, NHID), f32),
                        pltpu.VMEM((G, NP, NP), f32)],
    )(A0a, A0b, X0p, W1, b1.reshape(1, NHID), W2, b2.reshape(1, NHID),
      W3, b3.reshape(1, NHID))


E = 320000
GPG = 8
NGRP = 13
CH = 4000
NCHUNK = E // CH
HALF = NCHUNK // 2
AWORDS = GPG * NPG * NP
OUTW = NGRP * AWORDS


def _adj_sc_body(src_hbm, dst_hbm, out0_hbm, out1_hbm, acc, sb0, db0, sb1, db1,
                 sem0, sem1):
    wid = lax.axis_index("s") * 2 + lax.axis_index("c")
    role = wid % 2
    group = wid // 2
    zero16 = jnp.zeros((16,), jnp.float32)
    ones16 = jnp.ones((16,), jnp.float32)

    def zbody(i, c):
        for u in range(8):
            acc[pl.ds((i * 8 + u) * 16, 16)] = zero16
        return c

    lax.fori_loop(0, AWORDS // 128, zbody, 0)

    gbase = group * GPG
    cbase = role * HALF

    def process(sbuf, dbuf):
        def vec(v, cc):
            for u in range(5):
                s = sbuf[pl.ds((v * 5 + u) * 16, 16)]
                d = dbuf[pl.ds((v * 5 + u) * 16, 16)]
                g = lax.shift_right_logical(s * 5243, 19)
                srem = s - g * NPG
                trem = d - g * NPG
                lg = g - gbase
                own = (lg >= 0) & (lg < GPG)
                base = lg * (NPG * NP)
                idx1 = base + srem * NP + trem
                idx2 = base + trem * NP + srem
                plsc.addupdate_scatter(acc, [idx1], ones16, mask=own)
                plsc.addupdate_scatter(acc, [idx2], ones16, mask=own)
            return cc

        lax.fori_loop(0, CH // 80, vec, 0)

    def start(c, sbuf, dbuf):
        pltpu.make_async_copy(src_hbm.at[pl.ds(c * CH, CH)], sbuf, sem0).start()
        pltpu.make_async_copy(dst_hbm.at[pl.ds(c * CH, CH)], dbuf, sem1).start()

    def wait(sbuf, dbuf):
        pltpu.make_async_copy(src_hbm.at[pl.ds(0, CH)], sbuf, sem0).wait()
        pltpu.make_async_copy(dst_hbm.at[pl.ds(0, CH)], dbuf, sem1).wait()

    @pl.when(wid < 2 * NGRP)
    def _scan():
        start(cbase, sb0, db0)

        def pair(i, carry):
            c0 = cbase + i * 2
            wait(sb0, db0)
            start(c0 + 1, sb1, db1)
            process(sb0, db0)
            wait(sb1, db1)

            @pl.when(i + 1 < HALF // 2)
            def _():
                start(c0 + 2, sb0, db0)

            process(sb1, db1)
            return carry

        lax.fori_loop(0, HALF // 2, pair, 0)

        @pl.when(role == 0)
        def _():
            pltpu.sync_copy(acc, out0_hbm.at[pl.ds(group * AWORDS, AWORDS)])

        @pl.when(role == 1)
        def _():
            pltpu.sync_copy(acc, out1_hbm.at[pl.ds(group * AWORDS, AWORDS)])


def _adj_sc(edge_index):
    ei = edge_index.astype(jnp.int32)
    src, dst = ei[0], ei[1]
    mesh = plsc.VectorSubcoreMesh(core_axis_name="c", subcore_axis_name="s")
    k = functools.partial(
        pl.kernel,
        mesh=mesh,
        out_type=(jax.ShapeDtypeStruct((OUTW,), jnp.float32),
                  jax.ShapeDtypeStruct((OUTW,), jnp.float32)),
        scratch_types=[pltpu.VMEM((AWORDS,), jnp.float32),
                       pltpu.VMEM((CH,), jnp.int32),
                       pltpu.VMEM((CH,), jnp.int32),
                       pltpu.VMEM((CH,), jnp.int32),
                       pltpu.VMEM((CH,), jnp.int32),
                       pltpu.SemaphoreType.DMA,
                       pltpu.SemaphoreType.DMA],
        compiler_params=pltpu.CompilerParams(needs_layout_passes=False),
    )(_adj_sc_body)
    a, b = k(src, dst)
    a = a.reshape(NGRP * GPG, NPG, NP)[:G]
    b = b.reshape(NGRP * GPG, NPG, NP)[:G]
    return a, b


def kernel(x, edge_index, batch, W1, b1, W2, b2, W3, b3):
    A0a, A0b = _adj_sc(edge_index)
    X_pad = jnp.pad(x.reshape(G, NPG, D), ((0, 0), (0, NP - NPG), (0, 0)))

    X1p, X2p, X3p, summary = _megakernel(A0a, A0b, X_pad, W1, b1, W2, b2,
                                         W3, b3)

    xs0 = X1p[:, :NPG, :].reshape(-1, NHID)
    xs2 = X2p[:, :K1, :].reshape(-1, NHID)
    xs4 = X3p[:, :K2, :].reshape(-1, NHID)
    b0 = batch
    b2_ids = jnp.repeat(jnp.arange(G, dtype=jnp.int32), K1)
    b4_ids = jnp.repeat(jnp.arange(G, dtype=jnp.int32), K2)
    return (summary.reshape(G, 2 * NHID), xs0, xs2, xs4, b0, b2_ids, b4_ids)


# speedup vs baseline: 4.6745x; 1.0417x over previous
import functools
import jax, jax.numpy as jnp
from jax import lax
from jax.experimental import pallas as pl
from jax.experimental.pallas import tpu as pltpu
from jax.experimental.pallas import tpu_sc as plsc

G = 100
NPG = 100
NP = 128
N = G * NPG
D = 128
NHID = 128
K1, K1P = 50, 64
K2, K2P = 25, 32


def _eye(n):
    r = jax.lax.broadcasted_iota(jnp.int32, (n, n), 0)
    c = jax.lax.broadcasted_iota(jnp.int32, (n, n), 1)
    return jnp.where(r == c, 1.0, 0.0).astype(jnp.float32)


def _gcn_conv(A, X, W, brow, n, nreal):
    Ah = A + _eye(n)
    dc = jnp.sum(Ah, axis=1, keepdims=True)
    dr = jnp.sum(Ah, axis=0, keepdims=True)
    disc = 1.0 / jnp.sqrt(jnp.clip(dc, 1e-6))
    disr = 1.0 / jnp.sqrt(jnp.clip(dr, 1e-6))
    An = (jnp.broadcast_to(disc, (n, n)) * Ah) * jnp.broadcast_to(disr, (n, n))
    M = jax.lax.dot(An, X, preferred_element_type=jnp.float32)
    Y = jnp.maximum(jax.lax.dot(M, W, preferred_element_type=jnp.float32)
                    + jnp.broadcast_to(brow, (n, NHID)), 0.0)
    ri = jax.lax.broadcasted_iota(jnp.int32, (n, NHID), 0)
    return jnp.where(ri < nreal, Y, 0.0)


def _score_row(A, X, n):
    degc = jnp.clip(jnp.sum(A, axis=1, keepdims=True), 1.0)
    agg = jax.lax.dot(A, X, preferred_element_type=jnp.float32) \
        / jnp.broadcast_to(degc, (n, NHID))
    sc = jnp.sum(jnp.abs(X - agg), axis=1, keepdims=True)
    scT = jnp.transpose(jnp.broadcast_to(sc, (n, NHID)))
    return scT[0:1, :]


def _topk_ranks(scores, n, nreal, k):
    col = jax.lax.broadcasted_iota(jnp.int32, (G, n), 1)
    sc = jnp.where(col < nreal, scores, -1.0)
    rank0 = jnp.full((G, n), 999.0, dtype=jnp.float32)

    def body(j, carry):
        sc, rank = carry
        mx = jnp.max(sc, axis=1, keepdims=True)
        cand = sc == mx
        am = jnp.min(jnp.where(cand, col, n), axis=1, keepdims=True)
        oh = col == am
        rank = jnp.where(oh, j.astype(jnp.float32), rank)
        return jnp.where(oh, -2.0, sc), rank

    _, rank = jax.lax.fori_loop(0, k, body, (sc, rank0))
    return rank


def _P_from_rank(rankrow, kp, n):
    rk = jnp.broadcast_to(rankrow, (kp, n)).astype(jnp.int32)
    rowi = jax.lax.broadcasted_iota(jnp.int32, (kp, n), 0)
    return jnp.where(rk == rowi, 1.0, 0.0).astype(jnp.float32)


def _mega_body(A0a_ref, A0b_ref, X0_ref, W1_ref, b1_ref, W2_ref, b2_ref, W3_ref, b3_ref,
               X1_ref, X2_ref, X3_ref, sum_ref,
               sc1_ref, rk1_ref, Xp1_ref, A1_ref, sc2_ref, rk2_ref,
               Xp2_ref, A2_ref, M1_ref, M2_ref, M3_ref, A0_ref):
    W1, b1 = W1_ref[...], b1_ref[...]
    W2, b2 = W2_ref[...], b2_ref[...]
    W3, b3 = W3_ref[...], b3_ref[...]
    HI = jax.lax.Precision.HIGHEST
    UN = 8
    zpad = jnp.zeros((NP - NPG, NP), jnp.float32)

    def stage_a0(i, c):
        for u in range(UN):
            g = i * UN + u
            A0_ref[g] = jnp.concatenate([A0a_ref[g] + A0b_ref[g], zpad], axis=0)
        return c

    jax.lax.fori_loop(0, G // UN, stage_a0, 0)

    def _norm_adj(A, n):
        Ah = A + _eye(n)
        dc = jnp.sum(Ah, axis=1, keepdims=True)
        dr = jnp.sum(Ah, axis=0, keepdims=True)
        disc = 1.0 / jnp.sqrt(jnp.clip(dc, 1e-6))
        disr = 1.0 / jnp.sqrt(jnp.clip(dr, 1e-6))
        return (jnp.broadcast_to(disc, (n, n)) * Ah) * jnp.broadcast_to(disr, (n, n))

    def _batched_xw(M_ref, W, b, nper, nreal, rows):
        M = M_ref[...].reshape(rows, NHID)
        Y = jnp.maximum(jax.lax.dot(M, W, preferred_element_type=jnp.float32)
                        + jnp.broadcast_to(b, (rows, NHID)), 0.0)
        ri = jax.lax.broadcasted_iota(jnp.int32, (rows, NHID), 0)
        return jnp.where(ri % nper < nreal, Y, 0.0)

    def stage_a1(i, c):
        for u in range(UN):
            g = i * UN + u
            An = _norm_adj(A0_ref[g], NP)
            M1_ref[g] = jax.lax.dot(An, X0_ref[g],
                                    preferred_element_type=jnp.float32)
        return c

    jax.lax.fori_loop(0, G // UN, stage_a1, 0)
    X1_ref[...] = _batched_xw(M1_ref, W1, b1, NP, NPG, G * NP) \
        .reshape(G, NP, NHID)

    def stage_a2(i, c):
        for u in range(UN):
            g = i * UN + u
            sc1_ref[g] = _score_row(A0_ref[g], X1_ref[g], NP)
        return c

    jax.lax.fori_loop(0, G // UN, stage_a2, 0)
    rk1_ref[...] = _topk_ranks(sc1_ref[...].reshape(G, NP), NP, NPG, K1) \
        .reshape(G, 1, NP)

    def stage_b1(i, c):
        for u in range(UN):
            g = i * UN + u
            P = _P_from_rank(rk1_ref[g], K1P, NP)
            A = A0_ref[g]
            Xp = jax.lax.dot(P, X1_ref[g], precision=HI,
                             preferred_element_type=jnp.float32)
            Xp1_ref[g] = Xp
            Ar = jax.lax.dot(P, A, preferred_element_type=jnp.float32)
            A1 = jax.lax.dot_general(Ar, P, (((1,), (1,)), ((), ())),
                                     preferred_element_type=jnp.float32)
            A1_ref[g] = A1
            An = _norm_adj(A1, K1P)
            M2_ref[g] = jax.lax.dot(An, Xp, preferred_element_type=jnp.float32)
        return c

    jax.lax.fori_loop(0, G // UN, stage_b1, 0)
    X2_ref[...] = _batched_xw(M2_ref, W2, b2, K1P, K1, G * K1P) \
        .reshape(G, K1P, NHID)

    def stage_b2(i, c):
        for u in range(UN):
            g = i * UN + u
            sc2_ref[g] = _score_row(A1_ref[g], X2_ref[g], K1P)
        return c

    jax.lax.fori_loop(0, G // UN, stage_b2, 0)
    rk2_ref[...] = _topk_ranks(sc2_ref[...].reshape(G, K1P), K1P, K1, K2) \
        .reshape(G, 1, K1P)

    def stage_c1(i, c):
        for u in range(UN):
            g = i * UN + u
            P = _P_from_rank(rk2_ref[g], K2P, K1P)
            Xp = jax.lax.dot(P, X2_ref[g], precision=HI,
                             preferred_element_type=jnp.float32)
            Xp2_ref[g] = Xp
            Ar = jax.lax.dot(P, A1_ref[g], preferred_element_type=jnp.float32)
            A2 = jax.lax.dot_general(Ar, P, (((1,), (1,)), ((), ())),
                                     preferred_element_type=jnp.float32)
            A2_ref[g] = A2
            An = _norm_adj(A2, K2P)
            M3_ref[g] = jax.lax.dot(An, Xp, preferred_element_type=jnp.float32)
        return c

    jax.lax.fori_loop(0, G // UN, stage_c1, 0)
    X3_ref[...] = _batched_xw(M3_ref, W3, b3, K2P, K2, G * K2P) \
        .reshape(G, K2P, NHID)

    def stage_c2(i, c):
        for u in range(UN):
            g = i * UN + u
            Xp1 = Xp1_ref[g]
            Xp2 = Xp2_ref[g]
            X3 = X3_ref[g]
            mx1 = jnp.max(Xp1, axis=0, keepdims=True)
            mn1 = jnp.sum(Xp1, axis=0, keepdims=True) / float(K1)
            mx2 = jnp.max(Xp2, axis=0, keepdims=True)
            mn2 = jnp.sum(Xp2, axis=0, keepdims=True) / float(K2)
            mx3 = jnp.max(X3, axis=0, keepdims=True)
            mn3 = jnp.sum(X3, axis=0, keepdims=True) / float(K2)
            r = jnp.maximum
            smax = r(mx1, 0.) + r(mx2, 0.) + r(mx3, 0.)
            smean = r(mn1, 0.) + r(mn2, 0.) + r(mn3, 0.)
            sum_ref[g] = jnp.concatenate([smax, smean], axis=1)
        return c

    jax.lax.fori_loop(0, G // UN, stage_c2, 0)


def _megakernel(A0a, A0b, X0p, W1, b1, W2, b2, W3, b3):
    f32 = jnp.float32
    return pl.pallas_call(
        _mega_body,
        out_shape=(jax.ShapeDtypeStruct((G, NP, NHID), f32),
                   jax.ShapeDtypeStruct((G, K1P, NHID), f32),
                   jax.ShapeDtypeStruct((G, K2P, NHID), f32),
                   jax.ShapeDtypeStruct((G, 1, 2 * NHID), f32)),
        scratch_shapes=[pltpu.VMEM((G, 1, NP), f32),
                        pltpu.VMEM((G, 1, NP), f32),
                        pltpu.VMEM((G, K1P, NHID), f32),
                        pltpu.VMEM((G, K1P, K1P), f32),
                        pltpu.VMEM((G, 1, K1P), f32),
                        pltpu.VMEM((G, 1, K1P), f32),
                        pltpu.VMEM((G, K2P, NHID), f32),
                        pltpu.VMEM((G, K2P, K2P), f32),
                        pltpu.VMEM((G, NP, NHID), f32),
                        pltpu.VMEM((G, K1P, NHID), f32),
                        pltpu.VMEM((G, K2P, NHID), f32),
                        pltpu.VMEM((G, NP, NP), f32)],
    )(A0a, A0b, X0p, W1, b1.reshape(1, NHID), W2, b2.reshape(1, NHID),
      W3, b3.reshape(1, NHID))


E = 320000
GPG = 8
NGRP = 13
CH = 4000
NCHUNK = E // CH
HALF = NCHUNK // 2
AWORDS = GPG * NPG * NP
OUTW = NGRP * AWORDS


def _adj_sc_body(src_hbm, dst_hbm, out0_hbm, out1_hbm, acc, sb0, db0, sb1, db1,
                 sem0, sem1):
    wid = lax.axis_index("s") * 2 + lax.axis_index("c")
    role = wid % 2
    group = wid // 2
    zero16 = jnp.zeros((16,), jnp.float32)
    ones16 = jnp.ones((16,), jnp.float32)

    def zbody(i, c):
        for u in range(8):
            acc[pl.ds((i * 8 + u) * 16, 16)] = zero16
        return c

    lax.fori_loop(0, AWORDS // 128, zbody, 0)

    gbase = group * GPG
    cbase = role * HALF

    def process(sbuf, dbuf):
        def vec(v, cc):
            for u in range(5):
                s = sbuf[pl.ds((v * 5 + u) * 16, 16)]
                d = dbuf[pl.ds((v * 5 + u) * 16, 16)]
                g = lax.shift_right_logical(s * 5243, 19)
                srem = s - g * NPG
                trem = d - g * NPG
                lg = g - gbase
                own = (lg >= 0) & (lg < GPG)
                base = lg * (NPG * NP)
                idx1 = base + srem * NP + trem
                idx2 = base + trem * NP + srem
                plsc.addupdate_scatter(acc, [idx1], ones16, mask=own)
                plsc.addupdate_scatter(acc, [idx2], ones16, mask=own)
            return cc

        lax.fori_loop(0, CH // 80, vec, 0)

    def start(c, sbuf, dbuf):
        pltpu.make_async_copy(src_hbm.at[pl.ds(c * CH, CH)], sbuf, sem0).start()
        pltpu.make_async_copy(dst_hbm.at[pl.ds(c * CH, CH)], dbuf, sem1).start()

    def wait(sbuf, dbuf):
        pltpu.make_async_copy(src_hbm.at[pl.ds(0, CH)], sbuf, sem0).wait()
        pltpu.make_async_copy(dst_hbm.at[pl.ds(0, CH)], dbuf, sem1).wait()

    @pl.when(wid < 2 * NGRP)
    def _scan():
        start(cbase, sb0, db0)

        def pair(i, carry):
            c0 = cbase + i * 2
            wait(sb0, db0)
            start(c0 + 1, sb1, db1)
            process(sb0, db0)
            wait(sb1, db1)

            @pl.when(i + 1 < HALF // 2)
            def _():
                start(c0 + 2, sb0, db0)

            process(sb1, db1)
            return carry

        lax.fori_loop(0, HALF // 2, pair, 0)

        @pl.when(role == 0)
        def _():
            pltpu.sync_copy(acc, out0_hbm.at[pl.ds(group * AWORDS, AWORDS)])

        @pl.when(role == 1)
        def _():
            pltpu.sync_copy(acc, out1_hbm.at[pl.ds(group * AWORDS, AWORDS)])


def _adj_sc(edge_index):
    ei = edge_index.astype(jnp.int32)
    src, dst = ei[0], ei[1]
    mesh = plsc.VectorSubcoreMesh(core_axis_name="c", subcore_axis_name="s")
    k = functools.partial(
        pl.kernel,
        mesh=mesh,
        out_type=(jax.ShapeDtypeStruct((OUTW,), jnp.float32),
                  jax.ShapeDtypeStruct((OUTW,), jnp.float32)),
        scratch_types=[pltpu.VMEM((AWORDS,), jnp.float32),
                       pltpu.VMEM((CH,), jnp.int32),
                       pltpu.VMEM((CH,), jnp.int32),
                       pltpu.VMEM((CH,), jnp.int32),
                       pltpu.VMEM((CH,), jnp.int32),
                       pltpu.SemaphoreType.DMA,
                       pltpu.SemaphoreType.DMA],
        compiler_params=pltpu.CompilerParams(needs_layout_passes=False),
    )(_adj_sc_body)
    a, b = k(src, dst)
    a = a.reshape(NGRP * GPG, NPG, NP)[:G]
    b = b.reshape(NGRP * GPG, NPG, NP)[:G]
    return a, b


def kernel(x, edge_index, batch, W1, b1, W2, b2, W3, b3):
    A0a, A0b = _adj_sc(edge_index)
    X_pad = jnp.pad(x.reshape(G, NPG, D), ((0, 0), (0, NP - NPG), (0, 0)))

    X1p, X2p, X3p, summary = _megakernel(A0a, A0b, X_pad, W1, b1, W2, b2,
                                         W3, b3)

    xs0 = X1p[:, :NPG, :].reshape(-1, NHID)
    xs2 = X2p[:, :K1, :].reshape(-1, NHID)
    xs4 = X3p[:, :K2, :].reshape(-1, NHID)
    b0 = batch
    b2_ids = jnp.repeat(jnp.arange(G, dtype=jnp.int32), K1)
    b4_ids = jnp.repeat(jnp.arange(G, dtype=jnp.int32), K2)
    return (summary.reshape(G, 2 * NHID), xs0, xs2, xs4, b0, b2_ids, b4_ids)
